# baseline (speedup 1.0000x reference)
import functools

import jax
import jax.numpy as jnp
from jax import lax
from jax.experimental import pallas as pl
from jax.experimental.pallas import tpu as pltpu
from jax.experimental.pallas import tpu_sc as plsc

D = 64
MAX_NORM = 1.0
V = 100000
B = 1024
L = 20
NC, NS, LANES = 2, 16, 16
NW = NC * NS
NF_W = (B * L) // NW
NB_W = B // NW
CHUNK = 128
NCHUNK = NF_W // CHUNK
NGROUP = NF_W // LANES


def _rsqrt_nr(s):
    i = plsc.bitcast(s, jnp.int32)
    i = jnp.int32(0x5F3759DF) - (i >> 1)
    y = plsc.bitcast(i, jnp.float32)
    for _ in range(3):
        y = y * (1.5 - 0.5 * s * y * y)
    return y


def _pool_body(table_hbm, idx_hbm, out_hbm, idx_v, rows_v, out_v, sem):
    wid = lax.axis_index("s") * NC + lax.axis_index("c")
    pltpu.sync_copy(idx_hbm.at[wid], idx_v)
    copies = [
        pltpu.async_copy(table_hbm.at[idx_v.at[j]],
                         rows_v.at[pl.ds(j * CHUNK, CHUNK)], sem)
        for j in range(NCHUNK)
    ]
    for c in copies:
        c.wait()

    def pool_row(bi, _):
        accs = [jnp.zeros((LANES,), jnp.float32) for _ in range(D // LANES)]
        rbase = bi * L
        for c in range(L):
            vs = [rows_v[rbase + c, pl.ds(k * LANES, LANES)]
                  for k in range(D // LANES)]
            ssq = vs[0] * vs[0]
            for k in range(1, D // LANES):
                ssq = ssq + vs[k] * vs[k]
            s = jnp.broadcast_to(jnp.sum(ssq), (LANES,))
            norm = s * _rsqrt_nr(s)
            scale = jnp.minimum(1.0, 1.0 / (norm + 1e-7))
            for k in range(D // LANES):
                accs[k] = accs[k] + vs[k] * scale
        for k in range(D // LANES):
            out_v[bi, pl.ds(k * LANES, LANES)] = accs[k] * (1.0 / L)
        return ()

    lax.fori_loop(0, NB_W, pool_row, ())
    pltpu.sync_copy(out_v, out_hbm.at[pl.ds(wid * NB_W, NB_W)])


def _pool(table, idx):
    mesh = plsc.VectorSubcoreMesh(core_axis_name="c", subcore_axis_name="s")
    return pl.kernel(
        _pool_body,
        out_type=jax.ShapeDtypeStruct((B, D), jnp.float32),
        mesh=mesh,
        compiler_params=pltpu.CompilerParams(
            needs_layout_passes=False, use_tc_tiling_on_sc=False),
        scratch_types=[
            pltpu.VMEM((NCHUNK, CHUNK), jnp.int32),
            pltpu.VMEM((NF_W, D), jnp.float32),
            pltpu.VMEM((NB_W, D), jnp.float32),
            pltpu.SemaphoreType.DMA,
        ],
    )(table, idx)


NV = 2048
GRID = (V + NV - 1) // NV


def _mm_body(x_ref, w_ref, b_ref, o_ref):
    o_ref[...] = lax.dot_general(
        x_ref[...], w_ref[...], (((1,), (1,)), ((), ())),
        preferred_element_type=jnp.float32,
    ) + b_ref[...]


def _project(pooled, w, b2):
    return pl.pallas_call(
        _mm_body,
        grid=(GRID,),
        in_specs=[
            pl.BlockSpec((B, D), lambda i: (0, 0)),
            pl.BlockSpec((NV, D), lambda i: (i, 0)),
            pl.BlockSpec((1, NV), lambda i: (0, i)),
        ],
        out_specs=pl.BlockSpec((B, NV), lambda i: (0, i)),
        out_shape=jax.ShapeDtypeStruct((B, V), jnp.float32),
    )(pooled, w, b2)


def kernel(inputs_, table, W, b):
    idx = inputs_.astype(jnp.int32).reshape(NW, NCHUNK, CHUNK)
    pooled = _pool(table, idx)
    return _project(pooled, W, b.reshape(1, V))
